# baseline (speedup 1.0000x reference)
import functools
import jax
import jax.numpy as jnp
from jax import lax
from jax.experimental import pallas as pl
from jax.experimental.pallas import tpu as pltpu
from jax.experimental.pallas import tpu_sc as plsc

N = 10000
E = 160000
NCLASS = 64
E_REAL = E + N
NP = 10240
NPT = NP // 16
CHUNK = 128
NCHUNK = 84
EPT = NCHUNK * CHUNK
E_PAD = EPT * 16
HC = NCHUNK // 2
SCH = 96
NSC = EPT // SCH

BM = 1000
BM2 = 1280

_SC_PARAMS = pltpu.CompilerParams(needs_layout_passes=False)


def _mm_attn_body(x_ref, w_ref, a2_ref, hlo_ref, hhi_ref, as_ref, ad_ref,
                  mb_ref):
    i = pl.program_id(0)
    h = jnp.dot(x_ref[...], w_ref[...], preferred_element_type=jnp.float32)
    half = h.shape[1] // 2
    hlo_ref[...] = h[:, :half]
    hhi_ref[...] = h[:, half:]
    asad = jnp.dot(h, a2_ref[...], preferred_element_type=jnp.float32)
    as_ref[...] = asad[:, :1]
    ad_ref[...] = asad[:, 1:]
    cur = jnp.max(asad[:, 0]) + jnp.max(asad[:, 1])
    prev = jnp.where(i == 0, -3e38, mb_ref[...][0, 0])
    mb_ref[...] = jnp.maximum(prev, cur).reshape(1, 1)


def _mm_attn(x, W, A2):
    n, k = x.shape
    m = W.shape[1]
    half = m // 2
    return pl.pallas_call(
        _mm_attn_body,
        grid=(n // BM,),
        in_specs=[
            pl.BlockSpec((BM, k), lambda i: (i, 0)),
            pl.BlockSpec((k, m), lambda i: (0, 0)),
            pl.BlockSpec((k, 2), lambda i: (0, 0)),
        ],
        out_specs=[
            pl.BlockSpec((BM, half), lambda i: (i, 0)),
            pl.BlockSpec((BM, half), lambda i: (i, 0)),
            pl.BlockSpec((BM, 1), lambda i: (i, 0)),
            pl.BlockSpec((BM, 1), lambda i: (i, 0)),
            pl.BlockSpec((1, 1), lambda i: (0, 0)),
        ],
        out_shape=[
            jax.ShapeDtypeStruct((n, half), jnp.float32),
            jax.ShapeDtypeStruct((n, half), jnp.float32),
            jax.ShapeDtypeStruct((n, 1), jnp.float32),
            jax.ShapeDtypeStruct((n, 1), jnp.float32),
            jax.ShapeDtypeStruct((1, 1), jnp.float32),
        ],
    )(x, W, A2)


def _epi1_body(nlo_ref, nhi_ref, esum_ref, b1_ref, w2_ref, a22_ref,
               h2p_ref, as2_ref, ad2_ref, mb_ref):
    i = pl.program_id(0)
    num = jnp.concatenate([nlo_ref[...], nhi_ref[...]], axis=1)
    esum = jnp.sum(esum_ref[...], axis=0)[:, None]
    h = num / (esum + 1e-16) + b1_ref[...]
    h = jnp.where(h > 0, h, jnp.exp(jnp.minimum(h, 0.0)) - 1.0)
    h2 = jnp.dot(h, w2_ref[...], preferred_element_type=jnp.float32)
    h2p_ref[...] = jnp.concatenate(
        [h2, jnp.zeros_like(h2, shape=(h2.shape[0], 128 - h2.shape[1]))],
        axis=1)
    asad2 = jnp.dot(h2, a22_ref[...], preferred_element_type=jnp.float32)
    as2_ref[...] = asad2[:, :1]
    ad2_ref[...] = asad2[:, 1:]
    nbase = i * nlo_ref.shape[0]
    ridx = nbase + lax.broadcasted_iota(jnp.int32, (nlo_ref.shape[0],), 0)
    valid = ridx < N
    cur = (jnp.max(jnp.where(valid, asad2[:, 0], -3e38))
           + jnp.max(jnp.where(valid, asad2[:, 1], -3e38)))
    prev = jnp.where(i == 0, -3e38, mb_ref[...][0, 0])
    mb_ref[...] = jnp.maximum(prev, cur).reshape(1, 1)


def _epi1(num_lo, num_hi, esum_part, b1, W2, A22):
    n, halfk = num_lo.shape
    k = 2 * halfk
    m = W2.shape[1]
    half = m // 2
    return pl.pallas_call(
        _epi1_body,
        grid=(n // BM2,),
        in_specs=[
            pl.BlockSpec((BM2, halfk), lambda i: (i, 0)),
            pl.BlockSpec((BM2, halfk), lambda i: (i, 0)),
            pl.BlockSpec((2, BM2), lambda i: (0, i)),
            pl.BlockSpec((1, k), lambda i: (0, 0)),
            pl.BlockSpec((k, m), lambda i: (0, 0)),
            pl.BlockSpec((m, 2), lambda i: (0, 0)),
        ],
        out_specs=[
            pl.BlockSpec((BM2, 128), lambda i: (i, 0)),
            pl.BlockSpec((BM2, 1), lambda i: (i, 0)),
            pl.BlockSpec((BM2, 1), lambda i: (i, 0)),
            pl.BlockSpec((1, 1), lambda i: (0, 0)),
        ],
        out_shape=[
            jax.ShapeDtypeStruct((n, 128), jnp.float32),
            jax.ShapeDtypeStruct((n, 1), jnp.float32),
            jax.ShapeDtypeStruct((n, 1), jnp.float32),
            jax.ShapeDtypeStruct((1, 1), jnp.float32),
        ],
    )(num_lo, num_hi, esum_part, b1, W2, A22)


def _epi2_body(na_ref, nb_ref, esum_ref, b2_ref, out_ref):
    m = out_ref.shape[1]
    num = na_ref[...][:, :m] + nb_ref[...][:, :m]
    esum = jnp.sum(esum_ref[...], axis=0)[:, None]
    z = num / (esum + 1e-16) + b2_ref[...]
    zmax = jnp.max(z, axis=1, keepdims=True)
    zs = z - zmax
    lse = jnp.log(jnp.sum(jnp.exp(zs), axis=1, keepdims=True))
    out_ref[...] = zs - lse


def _epi2(num_a, num_b, esum_part, b2, m):
    n = num_a.shape[0]
    return pl.pallas_call(
        _epi2_body,
        grid=(n // BM2,),
        in_specs=[
            pl.BlockSpec((BM2, 128), lambda i: (i, 0)),
            pl.BlockSpec((BM2, 128), lambda i: (i, 0)),
            pl.BlockSpec((2, BM2), lambda i: (0, i)),
            pl.BlockSpec((1, m), lambda i: (0, 0)),
        ],
        out_specs=pl.BlockSpec((BM2, m), lambda i: (i, 0)),
        out_shape=jax.ShapeDtypeStruct((n, m), jnp.float32),
    )(num_a, num_b, esum_part, b2)


def _sc_weights(a_s, a_d, src3, dst3, mb16):
    n_tab = a_s.shape[0]
    mesh = plsc.VectorSubcoreMesh(core_axis_name="c", subcore_axis_name="s")

    @functools.partial(
        pl.kernel,
        out_type=[
            jax.ShapeDtypeStruct((32, HC, CHUNK), jnp.float32),
            jax.ShapeDtypeStruct((2, NP), jnp.float32),
        ],
        mesh=mesh,
        compiler_params=_SC_PARAMS,
        scratch_types=[
            pltpu.VMEM_SHARED((NP,), jnp.float32),
            pltpu.VMEM((n_tab,), jnp.float32),
            pltpu.VMEM((n_tab,), jnp.float32),
            pltpu.VMEM((HC, CHUNK), jnp.int32),
            pltpu.VMEM((HC, CHUNK), jnp.int32),
            pltpu.VMEM((HC, CHUNK), jnp.float32),
            pltpu.VMEM((NPT,), jnp.float32),
            pltpu.VMEM((16,), jnp.float32),
        ],
    )
    def k(as_h, ad_h, src_h, dst_h, mb_h, w3_h, esump_h,
          esum_sp, as_v, ad_v, src_v, dst_v, w_v, zbuf, mb_v):
        c = lax.axis_index("c")
        s = lax.axis_index("s")
        zero16 = jnp.zeros((16,), jnp.float32)
        iota16 = lax.iota(jnp.int32, 16)

        wid = s * 2 + c
        pltpu.sync_copy(as_h, as_v)
        pltpu.sync_copy(ad_h, ad_v)
        pltpu.sync_copy(mb_h, mb_v)
        pltpu.sync_copy(src_h.at[wid], src_v)
        pltpu.sync_copy(dst_h.at[wid], dst_v)
        mb = mb_v[...]

        def zb(i, _):
            zbuf[pl.ds(i * 16, 16)] = zero16
            return 0
        lax.fori_loop(0, NPT // 16, zb, 0)
        pltpu.sync_copy(zbuf, esum_sp.at[pl.ds(s * NPT, NPT)])
        plsc.subcore_barrier()

        ebase = wid * (HC * CHUNK)

        def grp(g, _):
            sv = src_v[g // 8, pl.ds((g % 8) * 16, 16)]
            dv = dst_v[g // 8, pl.ds((g % 8) * 16, 16)]
            e = plsc.load_gather(as_v, [sv]) + plsc.load_gather(ad_v, [dv])
            e = jnp.maximum(e, 0.2 * e)
            w16 = jnp.exp(e - mb)
            gid = ebase + g * 16 + iota16
            w16 = jnp.where(gid < E_REAL, w16, 0.0)
            w_v[g // 8, pl.ds((g % 8) * 16, 16)] = w16
            return 0
        lax.fori_loop(0, HC * (CHUNK // 16), grp, 0)

        def srow(r, _):
            pltpu.sync_copy(w_v.at[r], esum_sp.at[dst_v.at[r]], add=True)
            return 0
        lax.fori_loop(0, HC, srow, 0)
        pltpu.sync_copy(w_v, w3_h.at[wid])
        plsc.subcore_barrier()
        pltpu.sync_copy(esum_sp.at[pl.ds(s * NPT, NPT)],
                        esump_h.at[c, pl.ds(s * NPT, NPT)])

    return k(a_s, a_d, src3, dst3, mb16)


def _scatter_pipeline(src_h, dst_h, w_h, h_ref, accum_sh,
                      sidx, didx, widx, rows_g, rows_m,
                      gsem, isem, ssem, base, nchunk):
    pltpu.sync_copy(src_h.at[pl.ds(base, SCH)], sidx.at[0])
    pltpu.sync_copy(dst_h.at[pl.ds(base, SCH)], didx.at[0])
    pltpu.sync_copy(w_h.at[pl.ds(base, SCH)], widx.at[0])
    pltpu.async_copy(src_h.at[pl.ds(base + SCH, SCH)], sidx.at[1], isem)
    pltpu.async_copy(dst_h.at[pl.ds(base + SCH, SCH)], didx.at[1], isem)
    pltpu.async_copy(w_h.at[pl.ds(base + SCH, SCH)], widx.at[1], isem)
    pltpu.async_copy(h_ref.at[sidx.at[0]], rows_g.at[0], gsem)

    def step(kk, _):
        slot = lax.rem(kk, 4)
        par = lax.rem(kk, 2)

        @pl.when(kk < nchunk - 1)
        def _():
            nslot = lax.rem(kk + 1, 4)
            npar = lax.rem(kk + 1, 2)
            off = base + (kk + 1) * SCH
            pltpu.make_async_copy(
                src_h.at[pl.ds(off, SCH)], sidx.at[nslot], isem).wait()
            pltpu.make_async_copy(
                dst_h.at[pl.ds(off, SCH)], didx.at[nslot], isem).wait()
            pltpu.make_async_copy(
                w_h.at[pl.ds(off, SCH)], widx.at[nslot], isem).wait()
            pltpu.async_copy(h_ref.at[sidx.at[nslot]], rows_g.at[npar], gsem)

        @pl.when(kk < nchunk - 2)
        def _():
            slot2 = lax.rem(kk + 2, 4)
            off2 = base + (kk + 2) * SCH
            pltpu.async_copy(src_h.at[pl.ds(off2, SCH)], sidx.at[slot2], isem)
            pltpu.async_copy(dst_h.at[pl.ds(off2, SCH)], didx.at[slot2], isem)
            pltpu.async_copy(w_h.at[pl.ds(off2, SCH)], widx.at[slot2], isem)

        pltpu.make_async_copy(
            h_ref.at[sidx.at[slot]], rows_g.at[par], gsem).wait()

        @pl.when(kk > 0)
        def _():
            pslot = lax.rem(kk - 1, 4)
            pltpu.make_async_copy(
                rows_m, accum_sh.at[didx.at[pslot]], ssem).wait()

        def grp(g, _):
            w16 = widx[slot, pl.ds(g * 16, 16)]
            for lane in range(16):
                w_s = w16[lane]
                row = g * 16 + lane
                for j in range(8):
                    rows_m[row, pl.ds(j * 16, 16)] = (
                        rows_g[par, row, pl.ds(j * 16, 16)] * w_s)
            return 0
        lax.fori_loop(0, SCH // 16, grp, 0)

        pltpu.async_copy(rows_m, accum_sh.at[didx.at[slot]], ssem, add=True)
        return 0
    lax.fori_loop(0, nchunk, step, 0)
    lslot = lax.rem(nchunk - 1, 4)
    pltpu.make_async_copy(rows_m, accum_sh.at[didx.at[lslot]], ssem).wait()


def _scatter_scratch():
    return [
        pltpu.VMEM_SHARED((NP, 128), jnp.float32),
        pltpu.VMEM((4, SCH), jnp.int32),
        pltpu.VMEM((4, SCH), jnp.int32),
        pltpu.VMEM((4, SCH), jnp.float32),
        pltpu.VMEM((2, SCH, 128), jnp.float32),
        pltpu.VMEM((SCH, 128), jnp.float32),
        pltpu.SemaphoreType.DMA,
        pltpu.SemaphoreType.DMA,
        pltpu.SemaphoreType.DMA,
    ]


def _zero_accum(rows_m, accum_sh, s):
    zero16 = jnp.zeros((16,), jnp.float32)

    def zr(i, _):
        rows_m[i // 8, pl.ds((i % 8) * 16, 16)] = zero16
        return 0
    lax.fori_loop(0, SCH * 8, zr, 0)

    def za(q, _):
        pltpu.sync_copy(rows_m.at[pl.ds(0, 64)],
                        accum_sh.at[pl.ds(s * NPT + q * 64, 64)])
        return 0
    lax.fori_loop(0, NPT // 64, za, 0)


def _sc_scatter(h_lo, h_hi, src_f, dst_f, w_f):
    mesh = plsc.VectorSubcoreMesh(core_axis_name="c", subcore_axis_name="s")

    @functools.partial(
        pl.kernel,
        out_type=[
            jax.ShapeDtypeStruct((NP, 128), jnp.float32),
            jax.ShapeDtypeStruct((NP, 128), jnp.float32),
        ],
        mesh=mesh,
        compiler_params=_SC_PARAMS,
        scratch_types=_scatter_scratch(),
    )
    def k(hlo_h, hhi_h, src_h, dst_h, w_h, numlo_h, numhi_h,
          accum_sh, sidx, didx, widx, rows_g, rows_m, gsem, isem, ssem):
        c = lax.axis_index("c")
        s = lax.axis_index("s")
        _zero_accum(rows_m, accum_sh, s)
        plsc.subcore_barrier()

        def run(h_ref, num_ref):
            _scatter_pipeline(src_h, dst_h, w_h, h_ref, accum_sh,
                              sidx, didx, widx, rows_g, rows_m,
                              gsem, isem, ssem, s * EPT, NSC)
            plsc.subcore_barrier()
            pltpu.sync_copy(accum_sh.at[pl.ds(s * NPT, NPT)],
                            num_ref.at[pl.ds(s * NPT, NPT)])

        @pl.when(c == 0)
        def _():
            run(hlo_h, numlo_h)

        @pl.when(c == 1)
        def _():
            run(hhi_h, numhi_h)

    return k(h_lo, h_hi, src_f, dst_f, w_f)


def _sc_scatter_es(h_pad, src_f, dst_f, w_f):
    mesh = plsc.VectorSubcoreMesh(core_axis_name="c", subcore_axis_name="s")

    @functools.partial(
        pl.kernel,
        out_type=[
            jax.ShapeDtypeStruct((NP, 128), jnp.float32),
            jax.ShapeDtypeStruct((NP, 128), jnp.float32),
        ],
        mesh=mesh,
        compiler_params=_SC_PARAMS,
        scratch_types=_scatter_scratch(),
    )
    def k(h_h, src_h, dst_h, w_h, numa_h, numb_h,
          accum_sh, sidx, didx, widx, rows_g, rows_m, gsem, isem, ssem):
        c = lax.axis_index("c")
        s = lax.axis_index("s")
        wid = s * 2 + c
        _zero_accum(rows_m, accum_sh, s)
        plsc.subcore_barrier()

        _scatter_pipeline(src_h, dst_h, w_h, h_h, accum_sh,
                          sidx, didx, widx, rows_g, rows_m,
                          gsem, isem, ssem, wid * (EPT // 2), NSC // 2)
        plsc.subcore_barrier()

        @pl.when(c == 0)
        def _():
            pltpu.sync_copy(accum_sh.at[pl.ds(s * NPT, NPT)],
                            numa_h.at[pl.ds(s * NPT, NPT)])

        @pl.when(c == 1)
        def _():
            pltpu.sync_copy(accum_sh.at[pl.ds(s * NPT, NPT)],
                            numb_h.at[pl.ds(s * NPT, NPT)])

    return k(h_pad, src_f, dst_f, w_f)


@jax.jit
def kernel(x, edge_index, W1, a_src1, a_dst1, b1, W2, a_src2, a_dst2, b2):
    loop = jnp.arange(N, dtype=edge_index.dtype)
    src = jnp.concatenate([edge_index[0], loop]).astype(jnp.int32)
    dst = jnp.concatenate([edge_index[1], loop]).astype(jnp.int32)
    pad = jnp.zeros((E_PAD - E_REAL,), jnp.int32)
    src_p = jnp.concatenate([src, pad])
    dst_p = jnp.concatenate([dst, pad])
    src4 = src_p.reshape(32, HC, CHUNK)
    dst4 = dst_p.reshape(32, HC, CHUNK)

    A2 = jnp.concatenate(
        [a_src1.reshape(-1, 1), a_dst1.reshape(-1, 1)], axis=1)
    A22 = jnp.concatenate(
        [a_src2.reshape(-1, 1), a_dst2.reshape(-1, 1)], axis=1)

    h1lo, h1hi, as1, ad1, mb1 = _mm_attn(x, W1, A2)
    mbv1 = mb1.reshape(())
    mbv1 = jnp.maximum(mbv1, 0.2 * mbv1)
    w4_1, esum1 = _sc_weights(as1.reshape(N), ad1.reshape(N), src4, dst4,
                              jnp.broadcast_to(mbv1.reshape(1), (16,)))
    n1lo, n1hi = _sc_scatter(h1lo, h1hi, src_p, dst_p, w4_1.reshape(E_PAD))

    h2p, as2, ad2, mb2 = _epi1(n1lo, n1hi, esum1,
                               b1.reshape(1, -1), W2, A22)
    mbv2 = mb2.reshape(())
    mbv2 = jnp.maximum(mbv2, 0.2 * mbv2)
    w4_2, esum2 = _sc_weights(as2.reshape(NP), ad2.reshape(NP), src4, dst4,
                              jnp.broadcast_to(mbv2.reshape(1), (16,)))
    n2a, n2b = _sc_scatter_es(h2p, src_p, dst_p, w4_2.reshape(E_PAD))

    out = _epi2(n2a, n2b, esum2, b2.reshape(1, -1), NCLASS)
    return out[:N]


# speedup vs baseline: 1.6149x; 1.6149x over previous
import functools
import jax
import jax.numpy as jnp
from jax import lax
from jax.experimental import pallas as pl
from jax.experimental.pallas import tpu as pltpu
from jax.experimental.pallas import tpu_sc as plsc

N = 10000
E = 160000
NCLASS = 64
E_REAL = E + N
NP = 10240
NPT = NP // 16
CHUNK = 128
NCHUNK = 84
EPT = NCHUNK * CHUNK
E_PAD = EPT * 16
HC = NCHUNK // 2
SCH = 96
NSC = EPT // SCH

BM = 1000
BM2 = 1280

_SC_PARAMS = pltpu.CompilerParams(needs_layout_passes=False)


def _mm_attn_body(x_ref, w_ref, a2_ref, hlo_ref, hhi_ref, as_ref, ad_ref,
                  mb_ref):
    i = pl.program_id(0)
    h = jnp.dot(x_ref[...], w_ref[...], preferred_element_type=jnp.float32)
    half = h.shape[1] // 2
    hlo_ref[...] = h[:, :half]
    hhi_ref[...] = h[:, half:]
    asad = jnp.dot(h, a2_ref[...], preferred_element_type=jnp.float32)
    as_ref[...] = asad[:, :1]
    ad_ref[...] = asad[:, 1:]
    cur = jnp.max(asad[:, 0]) + jnp.max(asad[:, 1])
    prev = jnp.where(i == 0, -3e38, mb_ref[...][0, 0])
    mb_ref[...] = jnp.maximum(prev, cur).reshape(1, 1)


def _mm_attn(x, W, A2):
    n, k = x.shape
    m = W.shape[1]
    half = m // 2
    return pl.pallas_call(
        _mm_attn_body,
        grid=(n // BM,),
        in_specs=[
            pl.BlockSpec((BM, k), lambda i: (i, 0)),
            pl.BlockSpec((k, m), lambda i: (0, 0)),
            pl.BlockSpec((k, 2), lambda i: (0, 0)),
        ],
        out_specs=[
            pl.BlockSpec((BM, half), lambda i: (i, 0)),
            pl.BlockSpec((BM, half), lambda i: (i, 0)),
            pl.BlockSpec((BM, 1), lambda i: (i, 0)),
            pl.BlockSpec((BM, 1), lambda i: (i, 0)),
            pl.BlockSpec((1, 1), lambda i: (0, 0)),
        ],
        out_shape=[
            jax.ShapeDtypeStruct((n, half), jnp.float32),
            jax.ShapeDtypeStruct((n, half), jnp.float32),
            jax.ShapeDtypeStruct((n, 1), jnp.float32),
            jax.ShapeDtypeStruct((n, 1), jnp.float32),
            jax.ShapeDtypeStruct((1, 1), jnp.float32),
        ],
    )(x, W, A2)


def _epi1_body(nlo_ref, nhi_ref, esum_ref, b1_ref, w2_ref, a22_ref,
               h2p_ref, as2_ref, ad2_ref, mb_ref):
    i = pl.program_id(0)
    num = jnp.concatenate([nlo_ref[...], nhi_ref[...]], axis=1)
    esum = jnp.sum(esum_ref[...], axis=0)[:, None]
    h = num / (esum + 1e-16) + b1_ref[...]
    h = jnp.where(h > 0, h, jnp.exp(jnp.minimum(h, 0.0)) - 1.0)
    h2 = jnp.dot(h, w2_ref[...], preferred_element_type=jnp.float32)
    h2p_ref[...] = jnp.concatenate(
        [h2, jnp.zeros_like(h2, shape=(h2.shape[0], 128 - h2.shape[1]))],
        axis=1)
    asad2 = jnp.dot(h2, a22_ref[...], preferred_element_type=jnp.float32)
    as2_ref[...] = asad2[:, :1]
    ad2_ref[...] = asad2[:, 1:]
    nbase = i * nlo_ref.shape[0]
    ridx = nbase + lax.broadcasted_iota(jnp.int32, (nlo_ref.shape[0],), 0)
    valid = ridx < N
    cur = (jnp.max(jnp.where(valid, asad2[:, 0], -3e38))
           + jnp.max(jnp.where(valid, asad2[:, 1], -3e38)))
    prev = jnp.where(i == 0, -3e38, mb_ref[...][0, 0])
    mb_ref[...] = jnp.maximum(prev, cur).reshape(1, 1)


def _epi1(num_lo, num_hi, esum_part, b1, W2, A22):
    n, halfk = num_lo.shape
    k = 2 * halfk
    m = W2.shape[1]
    half = m // 2
    return pl.pallas_call(
        _epi1_body,
        grid=(n // BM2,),
        in_specs=[
            pl.BlockSpec((BM2, halfk), lambda i: (i, 0)),
            pl.BlockSpec((BM2, halfk), lambda i: (i, 0)),
            pl.BlockSpec((2, BM2), lambda i: (0, i)),
            pl.BlockSpec((1, k), lambda i: (0, 0)),
            pl.BlockSpec((k, m), lambda i: (0, 0)),
            pl.BlockSpec((m, 2), lambda i: (0, 0)),
        ],
        out_specs=[
            pl.BlockSpec((BM2, 128), lambda i: (i, 0)),
            pl.BlockSpec((BM2, 1), lambda i: (i, 0)),
            pl.BlockSpec((BM2, 1), lambda i: (i, 0)),
            pl.BlockSpec((1, 1), lambda i: (0, 0)),
        ],
        out_shape=[
            jax.ShapeDtypeStruct((n, 128), jnp.float32),
            jax.ShapeDtypeStruct((n, 1), jnp.float32),
            jax.ShapeDtypeStruct((n, 1), jnp.float32),
            jax.ShapeDtypeStruct((1, 1), jnp.float32),
        ],
    )(num_lo, num_hi, esum_part, b1, W2, A22)


def _epi2_body(na_ref, nb_ref, esum_ref, b2_ref, out_ref):
    m = out_ref.shape[1]
    num = na_ref[...][:, :m] + nb_ref[...][:, :m]
    esum = jnp.sum(esum_ref[...], axis=0)[:, None]
    z = num / (esum + 1e-16) + b2_ref[...]
    zmax = jnp.max(z, axis=1, keepdims=True)
    zs = z - zmax
    lse = jnp.log(jnp.sum(jnp.exp(zs), axis=1, keepdims=True))
    out_ref[...] = zs - lse


def _epi2(num_a, num_b, esum_part, b2, m):
    n = num_a.shape[0]
    return pl.pallas_call(
        _epi2_body,
        grid=(n // BM2,),
        in_specs=[
            pl.BlockSpec((BM2, 128), lambda i: (i, 0)),
            pl.BlockSpec((BM2, 128), lambda i: (i, 0)),
            pl.BlockSpec((2, BM2), lambda i: (0, i)),
            pl.BlockSpec((1, m), lambda i: (0, 0)),
        ],
        out_specs=pl.BlockSpec((BM2, m), lambda i: (i, 0)),
        out_shape=jax.ShapeDtypeStruct((n, m), jnp.float32),
    )(num_a, num_b, esum_part, b2)


def _sc_weights(a_s, a_d, src3, dst3, mb16):
    n_tab = a_s.shape[0]
    mesh = plsc.VectorSubcoreMesh(core_axis_name="c", subcore_axis_name="s")

    @functools.partial(
        pl.kernel,
        out_type=[
            jax.ShapeDtypeStruct((32, HC, CHUNK), jnp.float32),
            jax.ShapeDtypeStruct((2, NP), jnp.float32),
        ],
        mesh=mesh,
        compiler_params=_SC_PARAMS,
        scratch_types=[
            pltpu.VMEM_SHARED((NP,), jnp.float32),
            pltpu.VMEM((n_tab,), jnp.float32),
            pltpu.VMEM((n_tab,), jnp.float32),
            pltpu.VMEM((HC, CHUNK), jnp.int32),
            pltpu.VMEM((HC, CHUNK), jnp.int32),
            pltpu.VMEM((HC, CHUNK), jnp.float32),
            pltpu.VMEM((NPT,), jnp.float32),
            pltpu.VMEM((16,), jnp.float32),
        ],
    )
    def k(as_h, ad_h, src_h, dst_h, mb_h, w3_h, esump_h,
          esum_sp, as_v, ad_v, src_v, dst_v, w_v, zbuf, mb_v):
        c = lax.axis_index("c")
        s = lax.axis_index("s")
        zero16 = jnp.zeros((16,), jnp.float32)
        iota16 = lax.iota(jnp.int32, 16)

        wid = s * 2 + c
        pltpu.sync_copy(as_h, as_v)
        pltpu.sync_copy(ad_h, ad_v)
        pltpu.sync_copy(mb_h, mb_v)
        pltpu.sync_copy(src_h.at[wid], src_v)
        pltpu.sync_copy(dst_h.at[wid], dst_v)
        mb = mb_v[...]

        def zb(i, _):
            zbuf[pl.ds(i * 16, 16)] = zero16
            return 0
        lax.fori_loop(0, NPT // 16, zb, 0)
        pltpu.sync_copy(zbuf, esum_sp.at[pl.ds(s * NPT, NPT)])
        plsc.subcore_barrier()

        ebase = wid * (HC * CHUNK)

        def grp(g, _):
            sv = src_v[g // 8, pl.ds((g % 8) * 16, 16)]
            dv = dst_v[g // 8, pl.ds((g % 8) * 16, 16)]
            e = plsc.load_gather(as_v, [sv]) + plsc.load_gather(ad_v, [dv])
            e = jnp.maximum(e, 0.2 * e)
            w16 = jnp.exp(e - mb)
            gid = ebase + g * 16 + iota16
            w16 = jnp.where(gid < E_REAL, w16, 0.0)
            w_v[g // 8, pl.ds((g % 8) * 16, 16)] = w16
            return 0
        lax.fori_loop(0, HC * (CHUNK // 16), grp, 0)

        def srow(r, _):
            pltpu.sync_copy(w_v.at[r], esum_sp.at[dst_v.at[r]], add=True)
            return 0
        lax.fori_loop(0, HC, srow, 0)
        pltpu.sync_copy(w_v, w3_h.at[wid])
        plsc.subcore_barrier()
        pltpu.sync_copy(esum_sp.at[pl.ds(s * NPT, NPT)],
                        esump_h.at[c, pl.ds(s * NPT, NPT)])

    return k(a_s, a_d, src3, dst3, mb16)


def _scatter_pipeline(src_h, dst_h, w_h, h_ref, accum_sh,
                      sidx, didx, widx, rows_g, rows_m,
                      gsem, isem, ssem, base, nchunk):
    pltpu.sync_copy(src_h.at[pl.ds(base, SCH)], sidx.at[0])
    pltpu.sync_copy(dst_h.at[pl.ds(base, SCH)], didx.at[0])
    pltpu.sync_copy(w_h.at[pl.ds(base, SCH)], widx.at[0])
    pltpu.async_copy(src_h.at[pl.ds(base + SCH, SCH)], sidx.at[1], isem)
    pltpu.async_copy(dst_h.at[pl.ds(base + SCH, SCH)], didx.at[1], isem)
    pltpu.async_copy(w_h.at[pl.ds(base + SCH, SCH)], widx.at[1], isem)
    pltpu.async_copy(h_ref.at[sidx.at[0]], rows_g.at[0], gsem)

    def step(kk, _):
        slot = lax.rem(kk, 4)
        par = lax.rem(kk, 2)

        @pl.when(kk < nchunk - 1)
        def _():
            nslot = lax.rem(kk + 1, 4)
            npar = lax.rem(kk + 1, 2)
            off = base + (kk + 1) * SCH
            pltpu.make_async_copy(
                src_h.at[pl.ds(off, SCH)], sidx.at[nslot], isem).wait()
            pltpu.make_async_copy(
                dst_h.at[pl.ds(off, SCH)], didx.at[nslot], isem).wait()
            pltpu.make_async_copy(
                w_h.at[pl.ds(off, SCH)], widx.at[nslot], isem).wait()
            pltpu.async_copy(h_ref.at[sidx.at[nslot]], rows_g.at[npar], gsem)

        @pl.when(kk < nchunk - 2)
        def _():
            slot2 = lax.rem(kk + 2, 4)
            off2 = base + (kk + 2) * SCH
            pltpu.async_copy(src_h.at[pl.ds(off2, SCH)], sidx.at[slot2], isem)
            pltpu.async_copy(dst_h.at[pl.ds(off2, SCH)], didx.at[slot2], isem)
            pltpu.async_copy(w_h.at[pl.ds(off2, SCH)], widx.at[slot2], isem)

        pltpu.make_async_copy(
            h_ref.at[sidx.at[slot]], rows_g.at[par], gsem).wait()

        @pl.when(kk > 0)
        def _():
            pslot = lax.rem(kk - 1, 4)
            pltpu.make_async_copy(
                rows_m, accum_sh.at[didx.at[pslot]], ssem).wait()

        def grp(g, _):
            w16 = widx[slot, pl.ds(g * 16, 16)]
            prev = None
            for lane in range(16):
                row = g * 16 + lane
                vals = [rows_g[par, row, pl.ds(j * 16, 16)] for j in range(8)]
                if prev is not None:
                    pv, prow, pw = prev
                    for j in range(8):
                        rows_m[prow, pl.ds(j * 16, 16)] = pv[j] * pw
                prev = (vals, row, w16[lane])
            pv, prow, pw = prev
            for j in range(8):
                rows_m[prow, pl.ds(j * 16, 16)] = pv[j] * pw
            return 0
        lax.fori_loop(0, SCH // 16, grp, 0)

        pltpu.async_copy(rows_m, accum_sh.at[didx.at[slot]], ssem, add=True)
        return 0
    lax.fori_loop(0, nchunk, step, 0)
    lslot = lax.rem(nchunk - 1, 4)
    pltpu.make_async_copy(rows_m, accum_sh.at[didx.at[lslot]], ssem).wait()


def _scatter_scratch():
    return [
        pltpu.VMEM_SHARED((NP, 128), jnp.float32),
        pltpu.VMEM((4, SCH), jnp.int32),
        pltpu.VMEM((4, SCH), jnp.int32),
        pltpu.VMEM((4, SCH), jnp.float32),
        pltpu.VMEM((2, SCH, 128), jnp.float32),
        pltpu.VMEM((SCH, 128), jnp.float32),
        pltpu.SemaphoreType.DMA,
        pltpu.SemaphoreType.DMA,
        pltpu.SemaphoreType.DMA,
    ]


def _zero_accum(rows_m, accum_sh, s):
    zero16 = jnp.zeros((16,), jnp.float32)

    def zr(i, _):
        rows_m[i // 8, pl.ds((i % 8) * 16, 16)] = zero16
        return 0
    lax.fori_loop(0, SCH * 8, zr, 0)

    def za(q, _):
        pltpu.sync_copy(rows_m.at[pl.ds(0, 64)],
                        accum_sh.at[pl.ds(s * NPT + q * 64, 64)])
        return 0
    lax.fori_loop(0, NPT // 64, za, 0)


def _sc_scatter(h_lo, h_hi, src_f, dst_f, w_f):
    mesh = plsc.VectorSubcoreMesh(core_axis_name="c", subcore_axis_name="s")

    @functools.partial(
        pl.kernel,
        out_type=[
            jax.ShapeDtypeStruct((NP, 128), jnp.float32),
            jax.ShapeDtypeStruct((NP, 128), jnp.float32),
        ],
        mesh=mesh,
        compiler_params=_SC_PARAMS,
        scratch_types=_scatter_scratch(),
    )
    def k(hlo_h, hhi_h, src_h, dst_h, w_h, numlo_h, numhi_h,
          accum_sh, sidx, didx, widx, rows_g, rows_m, gsem, isem, ssem):
        c = lax.axis_index("c")
        s = lax.axis_index("s")
        _zero_accum(rows_m, accum_sh, s)
        plsc.subcore_barrier()

        def run(h_ref, num_ref):
            _scatter_pipeline(src_h, dst_h, w_h, h_ref, accum_sh,
                              sidx, didx, widx, rows_g, rows_m,
                              gsem, isem, ssem, s * EPT, NSC)
            plsc.subcore_barrier()
            pltpu.sync_copy(accum_sh.at[pl.ds(s * NPT, NPT)],
                            num_ref.at[pl.ds(s * NPT, NPT)])

        @pl.when(c == 0)
        def _():
            run(hlo_h, numlo_h)

        @pl.when(c == 1)
        def _():
            run(hhi_h, numhi_h)

    return k(h_lo, h_hi, src_f, dst_f, w_f)


def _sc_scatter_es(h_pad, src_f, dst_f, w_f):
    mesh = plsc.VectorSubcoreMesh(core_axis_name="c", subcore_axis_name="s")

    @functools.partial(
        pl.kernel,
        out_type=[
            jax.ShapeDtypeStruct((NP, 128), jnp.float32),
            jax.ShapeDtypeStruct((NP, 128), jnp.float32),
        ],
        mesh=mesh,
        compiler_params=_SC_PARAMS,
        scratch_types=_scatter_scratch(),
    )
    def k(h_h, src_h, dst_h, w_h, numa_h, numb_h,
          accum_sh, sidx, didx, widx, rows_g, rows_m, gsem, isem, ssem):
        c = lax.axis_index("c")
        s = lax.axis_index("s")
        wid = s * 2 + c
        _zero_accum(rows_m, accum_sh, s)
        plsc.subcore_barrier()

        _scatter_pipeline(src_h, dst_h, w_h, h_h, accum_sh,
                          sidx, didx, widx, rows_g, rows_m,
                          gsem, isem, ssem, wid * (EPT // 2), NSC // 2)
        plsc.subcore_barrier()

        @pl.when(c == 0)
        def _():
            pltpu.sync_copy(accum_sh.at[pl.ds(s * NPT, NPT)],
                            numa_h.at[pl.ds(s * NPT, NPT)])

        @pl.when(c == 1)
        def _():
            pltpu.sync_copy(accum_sh.at[pl.ds(s * NPT, NPT)],
                            numb_h.at[pl.ds(s * NPT, NPT)])

    return k(h_pad, src_f, dst_f, w_f)


@jax.jit
def kernel(x, edge_index, W1, a_src1, a_dst1, b1, W2, a_src2, a_dst2, b2):
    loop = jnp.arange(N, dtype=edge_index.dtype)
    src = jnp.concatenate([edge_index[0], loop]).astype(jnp.int32)
    dst = jnp.concatenate([edge_index[1], loop]).astype(jnp.int32)
    pad = jnp.zeros((E_PAD - E_REAL,), jnp.int32)
    src_p = jnp.concatenate([src, pad])
    dst_p = jnp.concatenate([dst, pad])
    src4 = src_p.reshape(32, HC, CHUNK)
    dst4 = dst_p.reshape(32, HC, CHUNK)

    A2 = jnp.concatenate(
        [a_src1.reshape(-1, 1), a_dst1.reshape(-1, 1)], axis=1)
    A22 = jnp.concatenate(
        [a_src2.reshape(-1, 1), a_dst2.reshape(-1, 1)], axis=1)

    h1lo, h1hi, as1, ad1, mb1 = _mm_attn(x, W1, A2)
    mbv1 = mb1.reshape(())
    mbv1 = jnp.maximum(mbv1, 0.2 * mbv1)
    w4_1, esum1 = _sc_weights(as1.reshape(N), ad1.reshape(N), src4, dst4,
                              jnp.broadcast_to(mbv1.reshape(1), (16,)))
    n1lo, n1hi = _sc_scatter(h1lo, h1hi, src_p, dst_p, w4_1.reshape(E_PAD))

    h2p, as2, ad2, mb2 = _epi1(n1lo, n1hi, esum1,
                               b1.reshape(1, -1), W2, A22)
    mbv2 = mb2.reshape(())
    mbv2 = jnp.maximum(mbv2, 0.2 * mbv2)
    w4_2, esum2 = _sc_weights(as2.reshape(NP), ad2.reshape(NP), src4, dst4,
                              jnp.broadcast_to(mbv2.reshape(1), (16,)))
    n2a, n2b = _sc_scatter_es(h2p, src_p, dst_p, w4_2.reshape(E_PAD))

    out = _epi2(n2a, n2b, esum2, b2.reshape(1, -1), NCLASS)
    return out[:N]


# speedup vs baseline: 1.6703x; 1.0343x over previous
import functools
import jax
import jax.numpy as jnp
from jax import lax
from jax.experimental import pallas as pl
from jax.experimental.pallas import tpu as pltpu
from jax.experimental.pallas import tpu_sc as plsc

N = 10000
E = 160000
NCLASS = 64
E_REAL = E + N
NP = 10240
NPT = NP // 16
CHUNK = 128
NCHUNK = 84
EPT = NCHUNK * CHUNK
E_PAD = EPT * 16
HC = NCHUNK // 2
SCH = 64
NSC = EPT // SCH

BM = 1000
BM2 = 1280

_SC_PARAMS = pltpu.CompilerParams(needs_layout_passes=False)


def _mm_attn_body(x_ref, w_ref, a2_ref, hlo_ref, hhi_ref, as_ref, ad_ref,
                  mb_ref):
    i = pl.program_id(0)
    h = jnp.dot(x_ref[...], w_ref[...], preferred_element_type=jnp.float32)
    half = h.shape[1] // 2
    hlo_ref[...] = h[:, :half]
    hhi_ref[...] = h[:, half:]
    asad = jnp.dot(h, a2_ref[...], preferred_element_type=jnp.float32)
    as_ref[...] = asad[:, :1]
    ad_ref[...] = asad[:, 1:]
    cur = jnp.max(asad[:, 0]) + jnp.max(asad[:, 1])
    prev = jnp.where(i == 0, -3e38, mb_ref[...][0, 0])
    mb_ref[...] = jnp.maximum(prev, cur).reshape(1, 1)


def _mm_attn(x, W, A2):
    n, k = x.shape
    m = W.shape[1]
    half = m // 2
    return pl.pallas_call(
        _mm_attn_body,
        grid=(n // BM,),
        in_specs=[
            pl.BlockSpec((BM, k), lambda i: (i, 0)),
            pl.BlockSpec((k, m), lambda i: (0, 0)),
            pl.BlockSpec((k, 2), lambda i: (0, 0)),
        ],
        out_specs=[
            pl.BlockSpec((BM, half), lambda i: (i, 0)),
            pl.BlockSpec((BM, half), lambda i: (i, 0)),
            pl.BlockSpec((BM, 1), lambda i: (i, 0)),
            pl.BlockSpec((BM, 1), lambda i: (i, 0)),
            pl.BlockSpec((1, 1), lambda i: (0, 0)),
        ],
        out_shape=[
            jax.ShapeDtypeStruct((n, half), jnp.float32),
            jax.ShapeDtypeStruct((n, half), jnp.float32),
            jax.ShapeDtypeStruct((n, 1), jnp.float32),
            jax.ShapeDtypeStruct((n, 1), jnp.float32),
            jax.ShapeDtypeStruct((1, 1), jnp.float32),
        ],
    )(x, W, A2)


def _epi1_body(nlo_ref, nhi_ref, esum_ref, b1_ref, w2_ref, a22_ref,
               h2p_ref, as2_ref, ad2_ref, mb_ref):
    i = pl.program_id(0)
    num = jnp.concatenate([nlo_ref[...], nhi_ref[...]], axis=1)
    esum = jnp.sum(esum_ref[...], axis=0)[:, None]
    h = num / (esum + 1e-16) + b1_ref[...]
    h = jnp.where(h > 0, h, jnp.exp(jnp.minimum(h, 0.0)) - 1.0)
    h2 = jnp.dot(h, w2_ref[...], preferred_element_type=jnp.float32)
    h2p_ref[...] = jnp.concatenate(
        [h2, jnp.zeros_like(h2, shape=(h2.shape[0], 128 - h2.shape[1]))],
        axis=1)
    asad2 = jnp.dot(h2, a22_ref[...], preferred_element_type=jnp.float32)
    as2_ref[...] = asad2[:, :1]
    ad2_ref[...] = asad2[:, 1:]
    nbase = i * nlo_ref.shape[0]
    ridx = nbase + lax.broadcasted_iota(jnp.int32, (nlo_ref.shape[0],), 0)
    valid = ridx < N
    cur = (jnp.max(jnp.where(valid, asad2[:, 0], -3e38))
           + jnp.max(jnp.where(valid, asad2[:, 1], -3e38)))
    prev = jnp.where(i == 0, -3e38, mb_ref[...][0, 0])
    mb_ref[...] = jnp.maximum(prev, cur).reshape(1, 1)


def _epi1(num_lo, num_hi, esum_part, b1, W2, A22):
    n, halfk = num_lo.shape
    k = 2 * halfk
    m = W2.shape[1]
    half = m // 2
    return pl.pallas_call(
        _epi1_body,
        grid=(n // BM2,),
        in_specs=[
            pl.BlockSpec((BM2, halfk), lambda i: (i, 0)),
            pl.BlockSpec((BM2, halfk), lambda i: (i, 0)),
            pl.BlockSpec((2, BM2), lambda i: (0, i)),
            pl.BlockSpec((1, k), lambda i: (0, 0)),
            pl.BlockSpec((k, m), lambda i: (0, 0)),
            pl.BlockSpec((m, 2), lambda i: (0, 0)),
        ],
        out_specs=[
            pl.BlockSpec((BM2, 128), lambda i: (i, 0)),
            pl.BlockSpec((BM2, 1), lambda i: (i, 0)),
            pl.BlockSpec((BM2, 1), lambda i: (i, 0)),
            pl.BlockSpec((1, 1), lambda i: (0, 0)),
        ],
        out_shape=[
            jax.ShapeDtypeStruct((n, 128), jnp.float32),
            jax.ShapeDtypeStruct((n, 1), jnp.float32),
            jax.ShapeDtypeStruct((n, 1), jnp.float32),
            jax.ShapeDtypeStruct((1, 1), jnp.float32),
        ],
    )(num_lo, num_hi, esum_part, b1, W2, A22)


def _epi2_body(na_ref, nb_ref, esum_ref, b2_ref, out_ref):
    m = out_ref.shape[1]
    num = na_ref[...][:, :m] + nb_ref[...][:, :m]
    esum = jnp.sum(esum_ref[...], axis=0)[:, None]
    z = num / (esum + 1e-16) + b2_ref[...]
    zmax = jnp.max(z, axis=1, keepdims=True)
    zs = z - zmax
    lse = jnp.log(jnp.sum(jnp.exp(zs), axis=1, keepdims=True))
    out_ref[...] = zs - lse


def _epi2(num_a, num_b, esum_part, b2, m):
    n = num_a.shape[0]
    return pl.pallas_call(
        _epi2_body,
        grid=(n // BM2,),
        in_specs=[
            pl.BlockSpec((BM2, 128), lambda i: (i, 0)),
            pl.BlockSpec((BM2, 128), lambda i: (i, 0)),
            pl.BlockSpec((2, BM2), lambda i: (0, i)),
            pl.BlockSpec((1, m), lambda i: (0, 0)),
        ],
        out_specs=pl.BlockSpec((BM2, m), lambda i: (i, 0)),
        out_shape=jax.ShapeDtypeStruct((n, m), jnp.float32),
    )(num_a, num_b, esum_part, b2)


def _sc_weights(a_s, a_d, src3, dst3, mb16):
    n_tab = a_s.shape[0]
    mesh = plsc.VectorSubcoreMesh(core_axis_name="c", subcore_axis_name="s")

    @functools.partial(
        pl.kernel,
        out_type=[
            jax.ShapeDtypeStruct((32, HC, CHUNK), jnp.float32),
            jax.ShapeDtypeStruct((2, NP), jnp.float32),
        ],
        mesh=mesh,
        compiler_params=_SC_PARAMS,
        scratch_types=[
            pltpu.VMEM_SHARED((NP,), jnp.float32),
            pltpu.VMEM((n_tab,), jnp.float32),
            pltpu.VMEM((n_tab,), jnp.float32),
            pltpu.VMEM((HC, CHUNK), jnp.int32),
            pltpu.VMEM((HC, CHUNK), jnp.int32),
            pltpu.VMEM((HC, CHUNK), jnp.float32),
            pltpu.VMEM((NPT,), jnp.float32),
            pltpu.VMEM((16,), jnp.float32),
        ],
    )
    def k(as_h, ad_h, src_h, dst_h, mb_h, w3_h, esump_h,
          esum_sp, as_v, ad_v, src_v, dst_v, w_v, zbuf, mb_v):
        c = lax.axis_index("c")
        s = lax.axis_index("s")
        zero16 = jnp.zeros((16,), jnp.float32)
        iota16 = lax.iota(jnp.int32, 16)

        wid = s * 2 + c
        pltpu.sync_copy(as_h, as_v)
        pltpu.sync_copy(ad_h, ad_v)
        pltpu.sync_copy(mb_h, mb_v)
        pltpu.sync_copy(src_h.at[wid], src_v)
        pltpu.sync_copy(dst_h.at[wid], dst_v)
        mb = mb_v[...]

        def zb(i, _):
            zbuf[pl.ds(i * 16, 16)] = zero16
            return 0
        lax.fori_loop(0, NPT // 16, zb, 0)
        pltpu.sync_copy(zbuf, esum_sp.at[pl.ds(s * NPT, NPT)])
        plsc.subcore_barrier()

        ebase = wid * (HC * CHUNK)

        def grp(g, _):
            sv = src_v[g // 8, pl.ds((g % 8) * 16, 16)]
            dv = dst_v[g // 8, pl.ds((g % 8) * 16, 16)]
            e = plsc.load_gather(as_v, [sv]) + plsc.load_gather(ad_v, [dv])
            e = jnp.maximum(e, 0.2 * e)
            w16 = jnp.exp(e - mb)
            gid = ebase + g * 16 + iota16
            w16 = jnp.where(gid < E_REAL, w16, 0.0)
            w_v[g // 8, pl.ds((g % 8) * 16, 16)] = w16
            return 0
        lax.fori_loop(0, HC * (CHUNK // 16), grp, 0)

        def srow(r, _):
            pltpu.sync_copy(w_v.at[r], esum_sp.at[dst_v.at[r]], add=True)
            return 0
        lax.fori_loop(0, HC, srow, 0)
        pltpu.sync_copy(w_v, w3_h.at[wid])
        plsc.subcore_barrier()
        pltpu.sync_copy(esum_sp.at[pl.ds(s * NPT, NPT)],
                        esump_h.at[c, pl.ds(s * NPT, NPT)])

    return k(a_s, a_d, src3, dst3, mb16)


def _scatter_pipeline(src_h, dst_h, w_h, h_ref, accum_sh,
                      sidx, didx, widx, rows_g, rows_m,
                      gsem, isem, ssem, base, nchunk):
    pltpu.sync_copy(src_h.at[pl.ds(base, SCH)], sidx.at[0])
    pltpu.sync_copy(dst_h.at[pl.ds(base, SCH)], didx.at[0])
    pltpu.sync_copy(w_h.at[pl.ds(base, SCH)], widx.at[0])
    pltpu.async_copy(src_h.at[pl.ds(base + SCH, SCH)], sidx.at[1], isem)
    pltpu.async_copy(dst_h.at[pl.ds(base + SCH, SCH)], didx.at[1], isem)
    pltpu.async_copy(w_h.at[pl.ds(base + SCH, SCH)], widx.at[1], isem)
    pltpu.async_copy(h_ref.at[sidx.at[0]], rows_g.at[0], gsem)

    def step(kk, _):
        slot = lax.rem(kk, 4)
        par = lax.rem(kk, 2)

        @pl.when(kk < nchunk - 1)
        def _():
            nslot = lax.rem(kk + 1, 4)
            npar = lax.rem(kk + 1, 2)
            off = base + (kk + 1) * SCH
            pltpu.make_async_copy(
                src_h.at[pl.ds(off, SCH)], sidx.at[nslot], isem).wait()
            pltpu.make_async_copy(
                dst_h.at[pl.ds(off, SCH)], didx.at[nslot], isem).wait()
            pltpu.make_async_copy(
                w_h.at[pl.ds(off, SCH)], widx.at[nslot], isem).wait()
            pltpu.async_copy(h_ref.at[sidx.at[nslot]], rows_g.at[npar], gsem)

        @pl.when(kk >= 2)
        def _():
            pslot = lax.rem(kk - 2, 4)
            pltpu.make_async_copy(
                rows_m.at[par], accum_sh.at[didx.at[pslot]], ssem).wait()

        @pl.when(kk < nchunk - 2)
        def _():
            slot2 = lax.rem(kk + 2, 4)
            off2 = base + (kk + 2) * SCH
            pltpu.async_copy(src_h.at[pl.ds(off2, SCH)], sidx.at[slot2], isem)
            pltpu.async_copy(dst_h.at[pl.ds(off2, SCH)], didx.at[slot2], isem)
            pltpu.async_copy(w_h.at[pl.ds(off2, SCH)], widx.at[slot2], isem)

        pltpu.make_async_copy(
            h_ref.at[sidx.at[slot]], rows_g.at[par], gsem).wait()

        def grp(g, _):
            w16 = widx[slot, pl.ds(g * 16, 16)]
            prev = None
            for lane in range(16):
                row = g * 16 + lane
                vals = [rows_g[par, row, pl.ds(j * 16, 16)] for j in range(8)]
                if prev is not None:
                    pv, prow, pw = prev
                    for j in range(8):
                        rows_m[par, prow, pl.ds(j * 16, 16)] = pv[j] * pw
                prev = (vals, row, w16[lane])
            pv, prow, pw = prev
            for j in range(8):
                rows_m[par, prow, pl.ds(j * 16, 16)] = pv[j] * pw
            return 0
        lax.fori_loop(0, SCH // 16, grp, 0)

        pltpu.async_copy(rows_m.at[par], accum_sh.at[didx.at[slot]], ssem,
                         add=True)
        return 0
    lax.fori_loop(0, nchunk, step, 0)
    for tail in (nchunk - 2, nchunk - 1):
        pltpu.make_async_copy(
            rows_m.at[tail % 2], accum_sh.at[didx.at[tail % 4]], ssem).wait()


def _scatter_scratch():
    return [
        pltpu.VMEM_SHARED((NP, 128), jnp.float32),
        pltpu.VMEM((4, SCH), jnp.int32),
        pltpu.VMEM((4, SCH), jnp.int32),
        pltpu.VMEM((4, SCH), jnp.float32),
        pltpu.VMEM((2, SCH, 128), jnp.float32),
        pltpu.VMEM((2, SCH, 128), jnp.float32),
        pltpu.SemaphoreType.DMA,
        pltpu.SemaphoreType.DMA,
        pltpu.SemaphoreType.DMA,
    ]


def _zero_accum(rows_m, accum_sh, s):
    zero16 = jnp.zeros((16,), jnp.float32)

    def zr(i, _):
        rows_m[0, i // 8, pl.ds((i % 8) * 16, 16)] = zero16
        return 0
    lax.fori_loop(0, SCH * 8, zr, 0)

    def za(q, _):
        pltpu.sync_copy(rows_m.at[0],
                        accum_sh.at[pl.ds(s * NPT + q * SCH, SCH)])
        return 0
    lax.fori_loop(0, NPT // SCH, za, 0)


def _sc_scatter(h_lo, h_hi, src_f, dst_f, w_f):
    mesh = plsc.VectorSubcoreMesh(core_axis_name="c", subcore_axis_name="s")

    @functools.partial(
        pl.kernel,
        out_type=[
            jax.ShapeDtypeStruct((NP, 128), jnp.float32),
            jax.ShapeDtypeStruct((NP, 128), jnp.float32),
        ],
        mesh=mesh,
        compiler_params=_SC_PARAMS,
        scratch_types=_scatter_scratch(),
    )
    def k(hlo_h, hhi_h, src_h, dst_h, w_h, numlo_h, numhi_h,
          accum_sh, sidx, didx, widx, rows_g, rows_m, gsem, isem, ssem):
        c = lax.axis_index("c")
        s = lax.axis_index("s")
        _zero_accum(rows_m, accum_sh, s)
        plsc.subcore_barrier()

        def run(h_ref, num_ref):
            _scatter_pipeline(src_h, dst_h, w_h, h_ref, accum_sh,
                              sidx, didx, widx, rows_g, rows_m,
                              gsem, isem, ssem, s * EPT, NSC)
            plsc.subcore_barrier()
            pltpu.sync_copy(accum_sh.at[pl.ds(s * NPT, NPT)],
                            num_ref.at[pl.ds(s * NPT, NPT)])

        @pl.when(c == 0)
        def _():
            run(hlo_h, numlo_h)

        @pl.when(c == 1)
        def _():
            run(hhi_h, numhi_h)

    return k(h_lo, h_hi, src_f, dst_f, w_f)


def _sc_scatter_es(h_pad, src_f, dst_f, w_f):
    mesh = plsc.VectorSubcoreMesh(core_axis_name="c", subcore_axis_name="s")

    @functools.partial(
        pl.kernel,
        out_type=[
            jax.ShapeDtypeStruct((NP, 128), jnp.float32),
            jax.ShapeDtypeStruct((NP, 128), jnp.float32),
        ],
        mesh=mesh,
        compiler_params=_SC_PARAMS,
        scratch_types=_scatter_scratch(),
    )
    def k(h_h, src_h, dst_h, w_h, numa_h, numb_h,
          accum_sh, sidx, didx, widx, rows_g, rows_m, gsem, isem, ssem):
        c = lax.axis_index("c")
        s = lax.axis_index("s")
        wid = s * 2 + c
        _zero_accum(rows_m, accum_sh, s)
        plsc.subcore_barrier()

        _scatter_pipeline(src_h, dst_h, w_h, h_h, accum_sh,
                          sidx, didx, widx, rows_g, rows_m,
                          gsem, isem, ssem, wid * (EPT // 2), NSC // 2)
        plsc.subcore_barrier()

        @pl.when(c == 0)
        def _():
            pltpu.sync_copy(accum_sh.at[pl.ds(s * NPT, NPT)],
                            numa_h.at[pl.ds(s * NPT, NPT)])

        @pl.when(c == 1)
        def _():
            pltpu.sync_copy(accum_sh.at[pl.ds(s * NPT, NPT)],
                            numb_h.at[pl.ds(s * NPT, NPT)])

    return k(h_pad, src_f, dst_f, w_f)


@jax.jit
def kernel(x, edge_index, W1, a_src1, a_dst1, b1, W2, a_src2, a_dst2, b2):
    loop = jnp.arange(N, dtype=edge_index.dtype)
    src = jnp.concatenate([edge_index[0], loop]).astype(jnp.int32)
    dst = jnp.concatenate([edge_index[1], loop]).astype(jnp.int32)
    pad = jnp.zeros((E_PAD - E_REAL,), jnp.int32)
    src_p = jnp.concatenate([src, pad])
    dst_p = jnp.concatenate([dst, pad])
    src4 = src_p.reshape(32, HC, CHUNK)
    dst4 = dst_p.reshape(32, HC, CHUNK)

    A2 = jnp.concatenate(
        [a_src1.reshape(-1, 1), a_dst1.reshape(-1, 1)], axis=1)
    A22 = jnp.concatenate(
        [a_src2.reshape(-1, 1), a_dst2.reshape(-1, 1)], axis=1)

    h1lo, h1hi, as1, ad1, mb1 = _mm_attn(x, W1, A2)
    mbv1 = mb1.reshape(())
    mbv1 = jnp.maximum(mbv1, 0.2 * mbv1)
    w4_1, esum1 = _sc_weights(as1.reshape(N), ad1.reshape(N), src4, dst4,
                              jnp.broadcast_to(mbv1.reshape(1), (16,)))
    n1lo, n1hi = _sc_scatter(h1lo, h1hi, src_p, dst_p, w4_1.reshape(E_PAD))

    h2p, as2, ad2, mb2 = _epi1(n1lo, n1hi, esum1,
                               b1.reshape(1, -1), W2, A22)
    mbv2 = mb2.reshape(())
    mbv2 = jnp.maximum(mbv2, 0.2 * mbv2)
    w4_2, esum2 = _sc_weights(as2.reshape(NP), ad2.reshape(NP), src4, dst4,
                              jnp.broadcast_to(mbv2.reshape(1), (16,)))
    n2a, n2b = _sc_scatter_es(h2p, src_p, dst_p, w4_2.reshape(E_PAD))

    out = _epi2(n2a, n2b, esum2, b2.reshape(1, -1), NCLASS)
    return out[:N]


# speedup vs baseline: 1.7347x; 1.0386x over previous
import functools
import jax
import jax.numpy as jnp
from jax import lax
from jax.experimental import pallas as pl
from jax.experimental.pallas import tpu as pltpu
from jax.experimental.pallas import tpu_sc as plsc

N = 10000
E = 160000
NCLASS = 64
E_REAL = E + N
NP = 10240
NPT = NP // 16
CHUNK = 128
NCHUNK = 84
EPT = NCHUNK * CHUNK
E_PAD = EPT * 16
HC = NCHUNK // 2
SCH = 64
NSC = EPT // SCH

BM = 1000
BM2 = 1280

_SC_PARAMS = pltpu.CompilerParams(needs_layout_passes=False)


def _mm_attn_body(x_ref, w_ref, a2_ref, hlo_ref, hhi_ref, as_ref, ad_ref,
                  mb_ref):
    i = pl.program_id(0)
    h = jnp.dot(x_ref[...], w_ref[...], preferred_element_type=jnp.float32)
    half = h.shape[1] // 2
    hlo_ref[...] = h[:, :half]
    hhi_ref[...] = h[:, half:]
    asad = jnp.dot(h, a2_ref[...], preferred_element_type=jnp.float32)
    as_ref[...] = asad[:, :1]
    ad_ref[...] = asad[:, 1:]
    cur = jnp.max(asad[:, 0]) + jnp.max(asad[:, 1])
    prev = jnp.where(i == 0, -3e38, mb_ref[...][0, 0])
    mb_ref[...] = jnp.maximum(prev, cur).reshape(1, 1)


def _mm_attn(x, W, A2):
    n, k = x.shape
    m = W.shape[1]
    half = m // 2
    return pl.pallas_call(
        _mm_attn_body,
        grid=(n // BM,),
        in_specs=[
            pl.BlockSpec((BM, k), lambda i: (i, 0)),
            pl.BlockSpec((k, m), lambda i: (0, 0)),
            pl.BlockSpec((k, 2), lambda i: (0, 0)),
        ],
        out_specs=[
            pl.BlockSpec((BM, half), lambda i: (i, 0)),
            pl.BlockSpec((BM, half), lambda i: (i, 0)),
            pl.BlockSpec((BM, 1), lambda i: (i, 0)),
            pl.BlockSpec((BM, 1), lambda i: (i, 0)),
            pl.BlockSpec((1, 1), lambda i: (0, 0)),
        ],
        out_shape=[
            jax.ShapeDtypeStruct((n, half), jnp.float32),
            jax.ShapeDtypeStruct((n, half), jnp.float32),
            jax.ShapeDtypeStruct((n, 1), jnp.float32),
            jax.ShapeDtypeStruct((n, 1), jnp.float32),
            jax.ShapeDtypeStruct((1, 1), jnp.float32),
        ],
    )(x, W, A2)


def _epi1_body(nlo_ref, nhi_ref, esum_ref, b1_ref, w2_ref, a22_ref,
               h2p_ref, as2_ref, ad2_ref, mb_ref):
    i = pl.program_id(0)
    num = jnp.concatenate([nlo_ref[...], nhi_ref[...]], axis=1)
    esum = jnp.sum(esum_ref[...], axis=0)[:, None]
    h = num / (esum + 1e-16) + b1_ref[...]
    h = jnp.where(h > 0, h, jnp.exp(jnp.minimum(h, 0.0)) - 1.0)
    h2 = jnp.dot(h, w2_ref[...], preferred_element_type=jnp.float32)
    h2p_ref[...] = jnp.concatenate(
        [h2, jnp.zeros_like(h2, shape=(h2.shape[0], 128 - h2.shape[1]))],
        axis=1)
    asad2 = jnp.dot(h2, a22_ref[...], preferred_element_type=jnp.float32)
    as2_ref[...] = asad2[:, :1]
    ad2_ref[...] = asad2[:, 1:]
    nbase = i * nlo_ref.shape[0]
    ridx = nbase + lax.broadcasted_iota(jnp.int32, (nlo_ref.shape[0],), 0)
    valid = ridx < N
    cur = (jnp.max(jnp.where(valid, asad2[:, 0], -3e38))
           + jnp.max(jnp.where(valid, asad2[:, 1], -3e38)))
    prev = jnp.where(i == 0, -3e38, mb_ref[...][0, 0])
    mb_ref[...] = jnp.maximum(prev, cur).reshape(1, 1)


def _epi1(num_lo, num_hi, esum_part, b1, W2, A22):
    n, halfk = num_lo.shape
    k = 2 * halfk
    m = W2.shape[1]
    half = m // 2
    return pl.pallas_call(
        _epi1_body,
        grid=(n // BM2,),
        in_specs=[
            pl.BlockSpec((BM2, halfk), lambda i: (i, 0)),
            pl.BlockSpec((BM2, halfk), lambda i: (i, 0)),
            pl.BlockSpec((2, BM2), lambda i: (0, i)),
            pl.BlockSpec((1, k), lambda i: (0, 0)),
            pl.BlockSpec((k, m), lambda i: (0, 0)),
            pl.BlockSpec((m, 2), lambda i: (0, 0)),
        ],
        out_specs=[
            pl.BlockSpec((BM2, 128), lambda i: (i, 0)),
            pl.BlockSpec((BM2, 1), lambda i: (i, 0)),
            pl.BlockSpec((BM2, 1), lambda i: (i, 0)),
            pl.BlockSpec((1, 1), lambda i: (0, 0)),
        ],
        out_shape=[
            jax.ShapeDtypeStruct((n, 128), jnp.float32),
            jax.ShapeDtypeStruct((n, 1), jnp.float32),
            jax.ShapeDtypeStruct((n, 1), jnp.float32),
            jax.ShapeDtypeStruct((1, 1), jnp.float32),
        ],
    )(num_lo, num_hi, esum_part, b1, W2, A22)


def _epi2_body(na_ref, nb_ref, esum_ref, b2_ref, out_ref):
    m = out_ref.shape[1]
    num = na_ref[...][:, :m] + nb_ref[...][:, :m]
    esum = jnp.sum(esum_ref[...], axis=0)[:, None]
    z = num / (esum + 1e-16) + b2_ref[...]
    zmax = jnp.max(z, axis=1, keepdims=True)
    zs = z - zmax
    lse = jnp.log(jnp.sum(jnp.exp(zs), axis=1, keepdims=True))
    out_ref[...] = zs - lse


def _epi2(num_a, num_b, esum_part, b2, m):
    n = num_a.shape[0]
    return pl.pallas_call(
        _epi2_body,
        grid=(n // BM2,),
        in_specs=[
            pl.BlockSpec((BM2, 128), lambda i: (i, 0)),
            pl.BlockSpec((BM2, 128), lambda i: (i, 0)),
            pl.BlockSpec((2, BM2), lambda i: (0, i)),
            pl.BlockSpec((1, m), lambda i: (0, 0)),
        ],
        out_specs=pl.BlockSpec((BM2, m), lambda i: (i, 0)),
        out_shape=jax.ShapeDtypeStruct((n, m), jnp.float32),
    )(num_a, num_b, esum_part, b2)


def _sc_weights(a_s, a_d, src3, dst3, mb16):
    n_tab = a_s.shape[0]
    mesh = plsc.VectorSubcoreMesh(core_axis_name="c", subcore_axis_name="s")

    @functools.partial(
        pl.kernel,
        out_type=[
            jax.ShapeDtypeStruct((32, HC, CHUNK), jnp.float32),
            jax.ShapeDtypeStruct((2, NP), jnp.float32),
        ],
        mesh=mesh,
        compiler_params=_SC_PARAMS,
        scratch_types=[
            pltpu.VMEM_SHARED((NP,), jnp.float32),
            pltpu.VMEM((n_tab,), jnp.float32),
            pltpu.VMEM((n_tab,), jnp.float32),
            pltpu.VMEM((HC, CHUNK), jnp.int32),
            pltpu.VMEM((HC, CHUNK), jnp.int32),
            pltpu.VMEM((HC, CHUNK), jnp.float32),
            pltpu.VMEM((NPT,), jnp.float32),
            pltpu.VMEM((16,), jnp.float32),
        ],
    )
    def k(as_h, ad_h, src_h, dst_h, mb_h, w3_h, esump_h,
          esum_sp, as_v, ad_v, src_v, dst_v, w_v, zbuf, mb_v):
        c = lax.axis_index("c")
        s = lax.axis_index("s")
        zero16 = jnp.zeros((16,), jnp.float32)
        iota16 = lax.iota(jnp.int32, 16)

        wid = s * 2 + c
        pltpu.sync_copy(as_h, as_v)
        pltpu.sync_copy(ad_h, ad_v)
        pltpu.sync_copy(mb_h, mb_v)
        pltpu.sync_copy(src_h.at[wid], src_v)
        pltpu.sync_copy(dst_h.at[wid], dst_v)
        mb = mb_v[...]

        def zb(i, _):
            zbuf[pl.ds(i * 16, 16)] = zero16
            return 0
        lax.fori_loop(0, NPT // 16, zb, 0)
        pltpu.sync_copy(zbuf, esum_sp.at[pl.ds(s * NPT, NPT)])
        plsc.subcore_barrier()

        ebase = wid * (HC * CHUNK)

        def grp(g, _):
            sv = src_v[g // 8, pl.ds((g % 8) * 16, 16)]
            dv = dst_v[g // 8, pl.ds((g % 8) * 16, 16)]
            e = plsc.load_gather(as_v, [sv]) + plsc.load_gather(ad_v, [dv])
            e = jnp.maximum(e, 0.2 * e)
            w16 = jnp.exp(e - mb)
            gid = ebase + g * 16 + iota16
            w16 = jnp.where(gid < E_REAL, w16, 0.0)
            w_v[g // 8, pl.ds((g % 8) * 16, 16)] = w16
            return 0
        lax.fori_loop(0, HC * (CHUNK // 16), grp, 0)

        def srow(r, _):
            pltpu.sync_copy(w_v.at[r], esum_sp.at[dst_v.at[r]], add=True)
            return 0
        lax.fori_loop(0, HC, srow, 0)
        pltpu.sync_copy(w_v, w3_h.at[wid])
        plsc.subcore_barrier()
        pltpu.sync_copy(esum_sp.at[pl.ds(s * NPT, NPT)],
                        esump_h.at[c, pl.ds(s * NPT, NPT)])

    return k(a_s, a_d, src3, dst3, mb16)


def _scatter_pipeline(src_h, dst_h, w_h, h_ref, accum_sh,
                      sidx, didx, widx, rows_g, rows_m,
                      gsem, isem, ssem, base, nchunk):
    def idx_copy(ck, fn):
        islot = lax.rem(ck, 5)
        off = base + ck * SCH
        fn(src_h.at[pl.ds(off, SCH)], sidx.at[islot])
        fn(dst_h.at[pl.ds(off, SCH)], didx.at[islot])
        fn(w_h.at[pl.ds(off, SCH)], widx.at[islot])

    idx_copy(0, pltpu.sync_copy)
    idx_copy(1, pltpu.sync_copy)
    idx_copy(2, lambda a, b: pltpu.async_copy(a, b, isem))
    pltpu.async_copy(h_ref.at[sidx.at[0]], rows_g.at[0], gsem)
    pltpu.async_copy(h_ref.at[sidx.at[1]], rows_g.at[1], gsem)

    def step(kk, _):
        slot = lax.rem(kk, 5)
        par3 = lax.rem(kk, 3)
        par2 = lax.rem(kk, 2)

        @pl.when(kk < nchunk - 2)
        def _():
            nslot = lax.rem(kk + 2, 5)
            off = base + (kk + 2) * SCH
            pltpu.make_async_copy(
                src_h.at[pl.ds(off, SCH)], sidx.at[nslot], isem).wait()
            pltpu.make_async_copy(
                dst_h.at[pl.ds(off, SCH)], didx.at[nslot], isem).wait()
            pltpu.make_async_copy(
                w_h.at[pl.ds(off, SCH)], widx.at[nslot], isem).wait()
            pltpu.async_copy(h_ref.at[sidx.at[nslot]],
                             rows_g.at[lax.rem(kk + 2, 3)], gsem)

        @pl.when(kk >= 2)
        def _():
            pslot = lax.rem(kk - 2, 5)
            pltpu.make_async_copy(
                rows_m.at[par2], accum_sh.at[didx.at[pslot]], ssem).wait()

        @pl.when(kk < nchunk - 3)
        def _():
            idx_copy(kk + 3, lambda a, b: pltpu.async_copy(a, b, isem))

        pltpu.make_async_copy(
            h_ref.at[sidx.at[slot]], rows_g.at[par3], gsem).wait()

        def grp(g, _):
            w16 = widx[slot, pl.ds(g * 16, 16)]
            prev = None
            for lane in range(16):
                row = g * 16 + lane
                vals = [rows_g[par3, row, pl.ds(j * 16, 16)]
                        for j in range(8)]
                if prev is not None:
                    pv, prow, pw = prev
                    for j in range(8):
                        rows_m[par2, prow, pl.ds(j * 16, 16)] = pv[j] * pw
                prev = (vals, row, w16[lane])
            pv, prow, pw = prev
            for j in range(8):
                rows_m[par2, prow, pl.ds(j * 16, 16)] = pv[j] * pw
            return 0
        lax.fori_loop(0, SCH // 16, grp, 0)

        pltpu.async_copy(rows_m.at[par2], accum_sh.at[didx.at[slot]], ssem,
                         add=True)
        return 0
    lax.fori_loop(0, nchunk, step, 0)
    for tail in (nchunk - 2, nchunk - 1):
        pltpu.make_async_copy(
            rows_m.at[tail % 2], accum_sh.at[didx.at[tail % 5]], ssem).wait()


def _scatter_scratch():
    return [
        pltpu.VMEM_SHARED((NP, 128), jnp.float32),
        pltpu.VMEM((5, SCH), jnp.int32),
        pltpu.VMEM((5, SCH), jnp.int32),
        pltpu.VMEM((5, SCH), jnp.float32),
        pltpu.VMEM((3, SCH, 128), jnp.float32),
        pltpu.VMEM((2, SCH, 128), jnp.float32),
        pltpu.SemaphoreType.DMA,
        pltpu.SemaphoreType.DMA,
        pltpu.SemaphoreType.DMA,
    ]


def _zero_accum(rows_m, accum_sh, s):
    zero16 = jnp.zeros((16,), jnp.float32)

    def zr(i, _):
        rows_m[0, i // 8, pl.ds((i % 8) * 16, 16)] = zero16
        return 0
    lax.fori_loop(0, SCH * 8, zr, 0)

    def za(q, _):
        pltpu.sync_copy(rows_m.at[0],
                        accum_sh.at[pl.ds(s * NPT + q * SCH, SCH)])
        return 0
    lax.fori_loop(0, NPT // SCH, za, 0)


def _sc_scatter(h_lo, h_hi, src_f, dst_f, w_f):
    mesh = plsc.VectorSubcoreMesh(core_axis_name="c", subcore_axis_name="s")

    @functools.partial(
        pl.kernel,
        out_type=[
            jax.ShapeDtypeStruct((NP, 128), jnp.float32),
            jax.ShapeDtypeStruct((NP, 128), jnp.float32),
        ],
        mesh=mesh,
        compiler_params=_SC_PARAMS,
        scratch_types=_scatter_scratch(),
    )
    def k(hlo_h, hhi_h, src_h, dst_h, w_h, numlo_h, numhi_h,
          accum_sh, sidx, didx, widx, rows_g, rows_m, gsem, isem, ssem):
        c = lax.axis_index("c")
        s = lax.axis_index("s")
        _zero_accum(rows_m, accum_sh, s)
        plsc.subcore_barrier()

        def run(h_ref, num_ref):
            _scatter_pipeline(src_h, dst_h, w_h, h_ref, accum_sh,
                              sidx, didx, widx, rows_g, rows_m,
                              gsem, isem, ssem, s * EPT, NSC)
            plsc.subcore_barrier()
            pltpu.sync_copy(accum_sh.at[pl.ds(s * NPT, NPT)],
                            num_ref.at[pl.ds(s * NPT, NPT)])

        @pl.when(c == 0)
        def _():
            run(hlo_h, numlo_h)

        @pl.when(c == 1)
        def _():
            run(hhi_h, numhi_h)

    return k(h_lo, h_hi, src_f, dst_f, w_f)


def _sc_scatter_es(h_pad, src_f, dst_f, w_f):
    mesh = plsc.VectorSubcoreMesh(core_axis_name="c", subcore_axis_name="s")

    @functools.partial(
        pl.kernel,
        out_type=[
            jax.ShapeDtypeStruct((NP, 128), jnp.float32),
            jax.ShapeDtypeStruct((NP, 128), jnp.float32),
        ],
        mesh=mesh,
        compiler_params=_SC_PARAMS,
        scratch_types=_scatter_scratch(),
    )
    def k(h_h, src_h, dst_h, w_h, numa_h, numb_h,
          accum_sh, sidx, didx, widx, rows_g, rows_m, gsem, isem, ssem):
        c = lax.axis_index("c")
        s = lax.axis_index("s")
        wid = s * 2 + c
        _zero_accum(rows_m, accum_sh, s)
        plsc.subcore_barrier()

        _scatter_pipeline(src_h, dst_h, w_h, h_h, accum_sh,
                          sidx, didx, widx, rows_g, rows_m,
                          gsem, isem, ssem, wid * (EPT // 2), NSC // 2)
        plsc.subcore_barrier()

        @pl.when(c == 0)
        def _():
            pltpu.sync_copy(accum_sh.at[pl.ds(s * NPT, NPT)],
                            numa_h.at[pl.ds(s * NPT, NPT)])

        @pl.when(c == 1)
        def _():
            pltpu.sync_copy(accum_sh.at[pl.ds(s * NPT, NPT)],
                            numb_h.at[pl.ds(s * NPT, NPT)])

    return k(h_pad, src_f, dst_f, w_f)


@jax.jit
def kernel(x, edge_index, W1, a_src1, a_dst1, b1, W2, a_src2, a_dst2, b2):
    loop = jnp.arange(N, dtype=edge_index.dtype)
    src = jnp.concatenate([edge_index[0], loop]).astype(jnp.int32)
    dst = jnp.concatenate([edge_index[1], loop]).astype(jnp.int32)
    pad = jnp.zeros((E_PAD - E_REAL,), jnp.int32)
    src_p = jnp.concatenate([src, pad])
    dst_p = jnp.concatenate([dst, pad])
    src4 = src_p.reshape(32, HC, CHUNK)
    dst4 = dst_p.reshape(32, HC, CHUNK)

    A2 = jnp.concatenate(
        [a_src1.reshape(-1, 1), a_dst1.reshape(-1, 1)], axis=1)
    A22 = jnp.concatenate(
        [a_src2.reshape(-1, 1), a_dst2.reshape(-1, 1)], axis=1)

    h1lo, h1hi, as1, ad1, mb1 = _mm_attn(x, W1, A2)
    mbv1 = mb1.reshape(())
    mbv1 = jnp.maximum(mbv1, 0.2 * mbv1)
    w4_1, esum1 = _sc_weights(as1.reshape(N), ad1.reshape(N), src4, dst4,
                              jnp.broadcast_to(mbv1.reshape(1), (16,)))
    n1lo, n1hi = _sc_scatter(h1lo, h1hi, src_p, dst_p, w4_1.reshape(E_PAD))

    h2p, as2, ad2, mb2 = _epi1(n1lo, n1hi, esum1,
                               b1.reshape(1, -1), W2, A22)
    mbv2 = mb2.reshape(())
    mbv2 = jnp.maximum(mbv2, 0.2 * mbv2)
    w4_2, esum2 = _sc_weights(as2.reshape(NP), ad2.reshape(NP), src4, dst4,
                              jnp.broadcast_to(mbv2.reshape(1), (16,)))
    n2a, n2b = _sc_scatter_es(h2p, src_p, dst_p, w4_2.reshape(E_PAD))

    out = _epi2(n2a, n2b, esum2, b2.reshape(1, -1), NCLASS)
    return out[:N]


# speedup vs baseline: 1.7348x; 1.0000x over previous
import functools
import jax
import jax.numpy as jnp
from jax import lax
from jax.experimental import pallas as pl
from jax.experimental.pallas import tpu as pltpu
from jax.experimental.pallas import tpu_sc as plsc

N = 10000
E = 160000
NCLASS = 64
E_REAL = E + N
NP = 10240
NPT = NP // 16
CHUNK = 128
NCHUNK = 84
EPT = NCHUNK * CHUNK
E_PAD = EPT * 16
HC = NCHUNK // 2
SCH = 64
NSC = EPT // SCH

BM = 1000
BM2 = 1280

_SC_PARAMS = pltpu.CompilerParams(needs_layout_passes=False)


def _mm_attn_body(x_ref, w_ref, a2_ref, hlo_ref, hhi_ref, as_ref, ad_ref,
                  mb_ref):
    i = pl.program_id(0)
    h = jnp.dot(x_ref[...], w_ref[...], preferred_element_type=jnp.float32)
    half = h.shape[1] // 2
    hlo_ref[...] = h[:, :half]
    hhi_ref[...] = h[:, half:]
    asad = jnp.dot(h, a2_ref[...], preferred_element_type=jnp.float32)
    as_ref[...] = asad[:, :1]
    ad_ref[...] = asad[:, 1:]
    cur = jnp.max(asad[:, 0]) + jnp.max(asad[:, 1])
    prev = jnp.where(i == 0, -3e38, mb_ref[...][0, 0])
    mb_ref[...] = jnp.maximum(prev, cur).reshape(1, 1)


def _mm_attn(x, W, A2):
    n, k = x.shape
    m = W.shape[1]
    half = m // 2
    return pl.pallas_call(
        _mm_attn_body,
        grid=(n // BM,),
        in_specs=[
            pl.BlockSpec((BM, k), lambda i: (i, 0)),
            pl.BlockSpec((k, m), lambda i: (0, 0)),
            pl.BlockSpec((k, 2), lambda i: (0, 0)),
        ],
        out_specs=[
            pl.BlockSpec((BM, half), lambda i: (i, 0)),
            pl.BlockSpec((BM, half), lambda i: (i, 0)),
            pl.BlockSpec((BM, 1), lambda i: (i, 0)),
            pl.BlockSpec((BM, 1), lambda i: (i, 0)),
            pl.BlockSpec((1, 1), lambda i: (0, 0)),
        ],
        out_shape=[
            jax.ShapeDtypeStruct((n, half), jnp.float32),
            jax.ShapeDtypeStruct((n, half), jnp.float32),
            jax.ShapeDtypeStruct((n, 1), jnp.float32),
            jax.ShapeDtypeStruct((n, 1), jnp.float32),
            jax.ShapeDtypeStruct((1, 1), jnp.float32),
        ],
    )(x, W, A2)


def _epi1_body(nlo_ref, nhi_ref, esum_ref, b1_ref, w2_ref, a22_ref,
               h2p_ref, as2_ref, ad2_ref, mb_ref):
    i = pl.program_id(0)
    num = jnp.concatenate([nlo_ref[...], nhi_ref[...]], axis=1)
    esum = jnp.sum(esum_ref[...], axis=0)[:, None]
    h = num / (esum + 1e-16) + b1_ref[...]
    h = jnp.where(h > 0, h, jnp.exp(jnp.minimum(h, 0.0)) - 1.0)
    h2 = jnp.dot(h, w2_ref[...], preferred_element_type=jnp.float32)
    h2p_ref[...] = jnp.concatenate(
        [h2, jnp.zeros_like(h2, shape=(h2.shape[0], 128 - h2.shape[1]))],
        axis=1)
    asad2 = jnp.dot(h2, a22_ref[...], preferred_element_type=jnp.float32)
    as2_ref[...] = asad2[:, :1]
    ad2_ref[...] = asad2[:, 1:]
    nbase = i * nlo_ref.shape[0]
    ridx = nbase + lax.broadcasted_iota(jnp.int32, (nlo_ref.shape[0],), 0)
    valid = ridx < N
    cur = (jnp.max(jnp.where(valid, asad2[:, 0], -3e38))
           + jnp.max(jnp.where(valid, asad2[:, 1], -3e38)))
    prev = jnp.where(i == 0, -3e38, mb_ref[...][0, 0])
    mb_ref[...] = jnp.maximum(prev, cur).reshape(1, 1)


def _epi1(num_lo, num_hi, esum_part, b1, W2, A22):
    n, halfk = num_lo.shape
    k = 2 * halfk
    m = W2.shape[1]
    half = m // 2
    return pl.pallas_call(
        _epi1_body,
        grid=(n // BM2,),
        in_specs=[
            pl.BlockSpec((BM2, halfk), lambda i: (i, 0)),
            pl.BlockSpec((BM2, halfk), lambda i: (i, 0)),
            pl.BlockSpec((2, BM2), lambda i: (0, i)),
            pl.BlockSpec((1, k), lambda i: (0, 0)),
            pl.BlockSpec((k, m), lambda i: (0, 0)),
            pl.BlockSpec((m, 2), lambda i: (0, 0)),
        ],
        out_specs=[
            pl.BlockSpec((BM2, 128), lambda i: (i, 0)),
            pl.BlockSpec((BM2, 1), lambda i: (i, 0)),
            pl.BlockSpec((BM2, 1), lambda i: (i, 0)),
            pl.BlockSpec((1, 1), lambda i: (0, 0)),
        ],
        out_shape=[
            jax.ShapeDtypeStruct((n, 128), jnp.float32),
            jax.ShapeDtypeStruct((n, 1), jnp.float32),
            jax.ShapeDtypeStruct((n, 1), jnp.float32),
            jax.ShapeDtypeStruct((1, 1), jnp.float32),
        ],
    )(num_lo, num_hi, esum_part, b1, W2, A22)


def _epi2_body(na_ref, nb_ref, esum_ref, b2_ref, out_ref):
    m = out_ref.shape[1]
    num = na_ref[...][:, :m] + nb_ref[...][:, :m]
    esum = jnp.sum(esum_ref[...], axis=0)[:, None]
    z = num / (esum + 1e-16) + b2_ref[...]
    zmax = jnp.max(z, axis=1, keepdims=True)
    zs = z - zmax
    lse = jnp.log(jnp.sum(jnp.exp(zs), axis=1, keepdims=True))
    out_ref[...] = zs - lse


def _epi2(num_a, num_b, esum_part, b2, m):
    n = num_a.shape[0]
    return pl.pallas_call(
        _epi2_body,
        grid=(n // BM2,),
        in_specs=[
            pl.BlockSpec((BM2, 128), lambda i: (i, 0)),
            pl.BlockSpec((BM2, 128), lambda i: (i, 0)),
            pl.BlockSpec((2, BM2), lambda i: (0, i)),
            pl.BlockSpec((1, m), lambda i: (0, 0)),
        ],
        out_specs=pl.BlockSpec((BM2, m), lambda i: (i, 0)),
        out_shape=jax.ShapeDtypeStruct((n, m), jnp.float32),
    )(num_a, num_b, esum_part, b2)


def _sc_weights(a_s, a_d, src3, dst3, mb16):
    n_tab = a_s.shape[0]
    mesh = plsc.VectorSubcoreMesh(core_axis_name="c", subcore_axis_name="s")

    @functools.partial(
        pl.kernel,
        out_type=[
            jax.ShapeDtypeStruct((32, HC, CHUNK), jnp.float32),
            jax.ShapeDtypeStruct((2, NP), jnp.float32),
        ],
        mesh=mesh,
        compiler_params=_SC_PARAMS,
        scratch_types=[
            pltpu.VMEM_SHARED((NP,), jnp.float32),
            pltpu.VMEM((n_tab,), jnp.float32),
            pltpu.VMEM((n_tab,), jnp.float32),
            pltpu.VMEM((HC, CHUNK), jnp.int32),
            pltpu.VMEM((HC, CHUNK), jnp.int32),
            pltpu.VMEM((HC, CHUNK), jnp.float32),
            pltpu.VMEM((NPT,), jnp.float32),
            pltpu.VMEM((16,), jnp.float32),
        ],
    )
    def k(as_h, ad_h, src_h, dst_h, mb_h, w3_h, esump_h,
          esum_sp, as_v, ad_v, src_v, dst_v, w_v, zbuf, mb_v):
        c = lax.axis_index("c")
        s = lax.axis_index("s")
        zero16 = jnp.zeros((16,), jnp.float32)
        iota16 = lax.iota(jnp.int32, 16)

        wid = s * 2 + c
        pltpu.sync_copy(as_h, as_v)
        pltpu.sync_copy(ad_h, ad_v)
        pltpu.sync_copy(mb_h, mb_v)
        pltpu.sync_copy(src_h.at[wid], src_v)
        pltpu.sync_copy(dst_h.at[wid], dst_v)
        mb = mb_v[...]

        def zb(i, _):
            zbuf[pl.ds(i * 16, 16)] = zero16
            return 0
        lax.fori_loop(0, NPT // 16, zb, 0)
        pltpu.sync_copy(zbuf, esum_sp.at[pl.ds(s * NPT, NPT)])
        plsc.subcore_barrier()

        ebase = wid * (HC * CHUNK)

        def grp(g, _):
            sv = src_v[g // 8, pl.ds((g % 8) * 16, 16)]
            dv = dst_v[g // 8, pl.ds((g % 8) * 16, 16)]
            e = plsc.load_gather(as_v, [sv]) + plsc.load_gather(ad_v, [dv])
            e = jnp.maximum(e, 0.2 * e)
            w16 = jnp.exp(e - mb)
            gid = ebase + g * 16 + iota16
            w16 = jnp.where(gid < E_REAL, w16, 0.0)
            w_v[g // 8, pl.ds((g % 8) * 16, 16)] = w16
            return 0
        lax.fori_loop(0, HC * (CHUNK // 16), grp, 0)

        def srow(r, _):
            pltpu.sync_copy(w_v.at[r], esum_sp.at[dst_v.at[r]], add=True)
            return 0
        lax.fori_loop(0, HC, srow, 0)
        pltpu.sync_copy(w_v, w3_h.at[wid])
        plsc.subcore_barrier()
        pltpu.sync_copy(esum_sp.at[pl.ds(s * NPT, NPT)],
                        esump_h.at[c, pl.ds(s * NPT, NPT)])

    return k(a_s, a_d, src3, dst3, mb16)


def _scatter_pipeline(src_h, dst_h, w_h, h_ref, accum_sh,
                      sidx, didx, widx, rows_g, rows_m,
                      gsem, isem, ssem, base, nchunk):
    def idx_copy(ck, fn):
        islot = lax.rem(ck, 5)
        off = base + ck * SCH
        fn(src_h.at[pl.ds(off, SCH)], sidx.at[islot])
        fn(dst_h.at[pl.ds(off, SCH)], didx.at[islot])
        fn(w_h.at[pl.ds(off, SCH)], widx.at[islot])

    idx_copy(0, pltpu.sync_copy)
    idx_copy(1, pltpu.sync_copy)
    idx_copy(2, lambda a, b: pltpu.async_copy(a, b, isem))
    pltpu.async_copy(h_ref.at[sidx.at[0]], rows_g.at[0], gsem)
    pltpu.async_copy(h_ref.at[sidx.at[1]], rows_g.at[1], gsem)

    def step(kk, _):
        slot = lax.rem(kk, 5)
        par3 = lax.rem(kk, 3)
        par2 = lax.rem(kk, 2)

        @pl.when(kk < nchunk - 2)
        def _():
            nslot = lax.rem(kk + 2, 5)
            off = base + (kk + 2) * SCH
            pltpu.make_async_copy(
                src_h.at[pl.ds(off, SCH)], sidx.at[nslot], isem).wait()
            pltpu.make_async_copy(
                dst_h.at[pl.ds(off, SCH)], didx.at[nslot], isem).wait()
            pltpu.make_async_copy(
                w_h.at[pl.ds(off, SCH)], widx.at[nslot], isem).wait()
            pltpu.async_copy(h_ref.at[sidx.at[nslot]],
                             rows_g.at[lax.rem(kk + 2, 3)], gsem)

        @pl.when(kk >= 2)
        def _():
            pslot = lax.rem(kk - 2, 5)
            pltpu.make_async_copy(
                rows_m.at[par2], accum_sh.at[didx.at[pslot]], ssem).wait()

        @pl.when(kk < nchunk - 3)
        def _():
            idx_copy(kk + 3, lambda a, b: pltpu.async_copy(a, b, isem))

        pltpu.make_async_copy(
            h_ref.at[sidx.at[slot]], rows_g.at[par3], gsem).wait()

        def grp(g, _):
            w16 = widx[slot, pl.ds(g * 16, 16)]
            prev = None
            for lane in range(16):
                row = g * 16 + lane
                vals = [rows_g[par3, row, pl.ds(j * 16, 16)]
                        for j in range(8)]
                if prev is not None:
                    pv, prow, pw = prev
                    for j in range(8):
                        rows_m[par2, prow, pl.ds(j * 16, 16)] = pv[j] * pw
                prev = (vals, row, w16[lane])
            pv, prow, pw = prev
            for j in range(8):
                rows_m[par2, prow, pl.ds(j * 16, 16)] = pv[j] * pw
            return 0
        lax.fori_loop(0, SCH // 16, grp, 0)

        pltpu.async_copy(rows_m.at[par2], accum_sh.at[didx.at[slot]], ssem,
                         add=True)
        return 0
    lax.fori_loop(0, nchunk, step, 0)
    for tail in (nchunk - 2, nchunk - 1):
        pltpu.make_async_copy(
            rows_m.at[tail % 2], accum_sh.at[didx.at[tail % 5]], ssem).wait()


def _scatter_scratch():
    return [
        pltpu.VMEM_SHARED((NP, 128), jnp.float32),
        pltpu.VMEM((5, SCH), jnp.int32),
        pltpu.VMEM((5, SCH), jnp.int32),
        pltpu.VMEM((5, SCH), jnp.float32),
        pltpu.VMEM((3, SCH, 128), jnp.float32),
        pltpu.VMEM((2, SCH, 128), jnp.float32),
        pltpu.SemaphoreType.DMA,
        pltpu.SemaphoreType.DMA,
        pltpu.SemaphoreType.DMA,
    ]


def _zero_accum(rows_m, accum_sh, s):
    zero16 = jnp.zeros((16,), jnp.float32)

    def zr(i, _):
        rows_m[0, i // 8, pl.ds((i % 8) * 16, 16)] = zero16
        return 0
    lax.fori_loop(0, SCH * 8, zr, 0)

    def za(q, _):
        pltpu.sync_copy(rows_m.at[0],
                        accum_sh.at[pl.ds(s * NPT + q * SCH, SCH)])
        return 0
    lax.fori_loop(0, NPT // SCH, za, 0)


def _sc_scatter(h_lo, h_hi, src_f, dst_f, w_f):
    mesh = plsc.VectorSubcoreMesh(core_axis_name="c", subcore_axis_name="s")

    @functools.partial(
        pl.kernel,
        out_type=[
            jax.ShapeDtypeStruct((NP, 128), jnp.float32),
            jax.ShapeDtypeStruct((NP, 128), jnp.float32),
        ],
        mesh=mesh,
        compiler_params=_SC_PARAMS,
        scratch_types=_scatter_scratch(),
    )
    def k(hlo_h, hhi_h, src_h, dst_h, w_h, numlo_h, numhi_h,
          accum_sh, sidx, didx, widx, rows_g, rows_m, gsem, isem, ssem):
        c = lax.axis_index("c")
        s = lax.axis_index("s")
        _zero_accum(rows_m, accum_sh, s)
        plsc.subcore_barrier()

        def run(h_ref, num_ref):
            _scatter_pipeline(src_h, dst_h, w_h, h_ref, accum_sh,
                              sidx, didx, widx, rows_g, rows_m,
                              gsem, isem, ssem, s * EPT, NSC)
            plsc.subcore_barrier()
            pltpu.sync_copy(accum_sh.at[pl.ds(s * NPT, NPT)],
                            num_ref.at[pl.ds(s * NPT, NPT)])

        @pl.when(c == 0)
        def _():
            run(hlo_h, numlo_h)

        @pl.when(c == 1)
        def _():
            run(hhi_h, numhi_h)

    return k(h_lo, h_hi, src_f, dst_f, w_f)


def _sc_scatter_es(h_pad, src_f, dst_f, w_f):
    mesh = plsc.VectorSubcoreMesh(core_axis_name="c", subcore_axis_name="s")

    @functools.partial(
        pl.kernel,
        out_type=[
            jax.ShapeDtypeStruct((NP, 128), jnp.float32),
            jax.ShapeDtypeStruct((NP, 128), jnp.float32),
        ],
        mesh=mesh,
        compiler_params=_SC_PARAMS,
        scratch_types=_scatter_scratch(),
    )
    def k(h_h, src_h, dst_h, w_h, numa_h, numb_h,
          accum_sh, sidx, didx, widx, rows_g, rows_m, gsem, isem, ssem):
        c = lax.axis_index("c")
        s = lax.axis_index("s")
        wid = s * 2 + c
        _zero_accum(rows_m, accum_sh, s)
        plsc.subcore_barrier()

        _scatter_pipeline(src_h, dst_h, w_h, h_h, accum_sh,
                          sidx, didx, widx, rows_g, rows_m,
                          gsem, isem, ssem, s * EPT + c * (EPT // 2),
                          NSC // 2)
        plsc.subcore_barrier()

        @pl.when(c == 0)
        def _():
            pltpu.sync_copy(accum_sh.at[pl.ds(s * NPT, NPT)],
                            numa_h.at[pl.ds(s * NPT, NPT)])

        @pl.when(c == 1)
        def _():
            pltpu.sync_copy(accum_sh.at[pl.ds(s * NPT, NPT)],
                            numb_h.at[pl.ds(s * NPT, NPT)])

    return k(h_pad, src_f, dst_f, w_f)


@jax.jit
def kernel(x, edge_index, W1, a_src1, a_dst1, b1, W2, a_src2, a_dst2, b2):
    loop = jnp.arange(N, dtype=edge_index.dtype)
    src = jnp.concatenate([edge_index[0], loop]).astype(jnp.int32)
    dst = jnp.concatenate([edge_index[1], loop]).astype(jnp.int32)
    pad = jnp.zeros((E_PAD - E_REAL,), jnp.int32)
    src_p = jnp.concatenate([src, pad])
    dst_p = jnp.concatenate([dst, pad])
    src4 = src_p.reshape(32, HC, CHUNK)
    dst4 = dst_p.reshape(32, HC, CHUNK)

    A2 = jnp.concatenate(
        [a_src1.reshape(-1, 1), a_dst1.reshape(-1, 1)], axis=1)
    A22 = jnp.concatenate(
        [a_src2.reshape(-1, 1), a_dst2.reshape(-1, 1)], axis=1)

    h1lo, h1hi, as1, ad1, mb1 = _mm_attn(x, W1, A2)
    mbv1 = mb1.reshape(())
    mbv1 = jnp.maximum(mbv1, 0.2 * mbv1)
    w4_1, esum1 = _sc_weights(as1.reshape(N), ad1.reshape(N), src4, dst4,
                              jnp.broadcast_to(mbv1.reshape(1), (16,)))
    n1lo, n1hi = _sc_scatter(h1lo, h1hi, src_p, dst_p, w4_1.reshape(E_PAD))

    h2p, as2, ad2, mb2 = _epi1(n1lo, n1hi, esum1,
                               b1.reshape(1, -1), W2, A22)
    mbv2 = mb2.reshape(())
    mbv2 = jnp.maximum(mbv2, 0.2 * mbv2)
    w4_2, esum2 = _sc_weights(as2.reshape(NP), ad2.reshape(NP), src4, dst4,
                              jnp.broadcast_to(mbv2.reshape(1), (16,)))
    n2a, n2b = _sc_scatter_es(h2p, src_p, dst_p, w4_2.reshape(E_PAD))

    out = _epi2(n2a, n2b, esum2, b2.reshape(1, -1), NCLASS)
    return out[:N]
